# baseline (speedup 1.0000x reference)
import functools

import jax
import jax.numpy as jnp
from jax import lax
from jax.experimental import pallas as pl
from jax.experimental.pallas import tpu as pltpu
from jax.experimental.pallas import tpu_sc as plsc

N = 10000
E = 320000
D = 128
NC = 2
NS = 16
NW = NC * NS
NP = 10240
RPT = NP // NS
EP = 327680
EPW = EP // NW
C = 128
NCH = EPW // C
SB = 8
NSB = NCH // SB

_mesh = plsc.VectorSubcoreMesh(core_axis_name="c", subcore_axis_name="s")


DW = 16


@functools.partial(
    pl.kernel,
    out_type=jax.ShapeDtypeStruct((NC, NP, DW), jnp.float32),
    mesh=_mesh,
    compiler_params=pltpu.CompilerParams(use_tc_tiling_on_sc=False),
    scratch_types=[
        pltpu.VMEM((NCH, C), jnp.int32),
        pltpu.VMEM((C, DW), jnp.float32),
        pltpu.VMEM_SHARED((NP, DW), jnp.float32),
    ],
)
def _deg_kernel(dst_hbm, ones_hbm, zeros_hbm, out_hbm, idx_v, ones_v, deg_sh):
    cid = lax.axis_index("c")
    sid = lax.axis_index("s")
    wid = sid * NC + cid
    pltpu.sync_copy(dst_hbm.at[wid], idx_v)
    pltpu.sync_copy(ones_hbm, ones_v)
    pltpu.sync_copy(zeros_hbm, deg_sh.at[pl.ds(sid * RPT, RPT)])
    plsc.subcore_barrier()

    def body(j, carry):
        pltpu.sync_copy(ones_v, deg_sh.at[idx_v.at[j]], add=True)
        return carry

    lax.fori_loop(0, NCH, body, 0)
    plsc.subcore_barrier()
    pltpu.sync_copy(deg_sh.at[pl.ds(sid * RPT, RPT)],
                    out_hbm.at[cid, pl.ds(sid * RPT, RPT)])


@functools.partial(
    pl.kernel,
    out_type=jax.ShapeDtypeStruct((NC, NP, D), jnp.float32),
    mesh=_mesh,
    scratch_types=[
        pltpu.VMEM((SB, C), jnp.int32),
        pltpu.VMEM((SB, C), jnp.int32),
        pltpu.VMEM((C, D), jnp.float32),
        pltpu.VMEM((C, D), jnp.float32),
        pltpu.VMEM_SHARED((NP, D), jnp.float32),
        pltpu.SemaphoreType.DMA,
        pltpu.SemaphoreType.DMA,
    ],
)
def _agg_kernel(feat_hbm, src_hbm, dst_hbm, zeros_hbm, out_hbm,
                src_v, dst_v, rows0_v, rows1_v, agg_sh, sem0, sem1):
    cid = lax.axis_index("c")
    sid = lax.axis_index("s")
    wid = sid * NC + cid
    pltpu.sync_copy(zeros_hbm, agg_sh.at[pl.ds(sid * RPT, RPT)])
    plsc.subcore_barrier()

    def step(j, rows_v, sem):
        pltpu.make_async_copy(feat_hbm.at[src_v.at[j]], rows_v, sem).wait()
        pltpu.sync_copy(rows_v, agg_sh.at[dst_v.at[j]], add=True)

        @pl.when(j + 2 < SB)
        def _():
            pltpu.async_copy(feat_hbm.at[src_v.at[j + 2]], rows_v, sem)

    def inner(j, carry):
        @pl.when(j % 2 == 0)
        def _():
            step(j, rows0_v, sem0)

        @pl.when(j % 2 == 1)
        def _():
            step(j, rows1_v, sem1)

        return carry

    def outer(s, carry):
        pltpu.sync_copy(src_hbm.at[wid, s], src_v)
        pltpu.sync_copy(dst_hbm.at[wid, s], dst_v)
        pltpu.async_copy(feat_hbm.at[src_v.at[0]], rows0_v, sem0)
        pltpu.async_copy(feat_hbm.at[src_v.at[1]], rows1_v, sem1)
        lax.fori_loop(0, SB, inner, 0)
        return carry

    lax.fori_loop(0, NSB, outer, 0)
    plsc.subcore_barrier()
    pltpu.sync_copy(agg_sh.at[pl.ds(sid * RPT, RPT)],
                    out_hbm.at[cid, pl.ds(sid * RPT, RPT)])


def _dinv_from(degp_ref):
    deg = degp_ref[0] + degp_ref[1]
    dinv = lax.rsqrt(jnp.maximum(deg, 1.0))
    return jnp.where(deg > 0, dinv, 0.0)


def _prescale_body(degp_ref, x_ref, xs_ref):
    xs_ref[...] = x_ref[...] * _dinv_from(degp_ref)


_prescale = pl.pallas_call(
    _prescale_body,
    out_shape=jax.ShapeDtypeStruct((N, D), jnp.float32),
)


def _mid_body(degp_ref, p_ref, w_ref, b_ref, out_ref):
    dinv = _dinv_from(degp_ref)
    agg = (p_ref[0][:N] + p_ref[1][:N]) * dinv
    z1 = jnp.dot(agg, w_ref[...], preferred_element_type=jnp.float32) + b_ref[...]
    out_ref[...] = jnp.maximum(z1, 0.0) * dinv


_mid = pl.pallas_call(
    _mid_body,
    out_shape=jax.ShapeDtypeStruct((N, D), jnp.float32),
)


def _fin_body(degp_ref, q_ref, w_ref, b_ref, z_ref, g_ref):
    dinv = _dinv_from(degp_ref)
    agg = (q_ref[0][:N] + q_ref[1][:N]) * dinv
    z2 = jnp.dot(agg, w_ref[...], preferred_element_type=jnp.float32) + b_ref[...]
    z_ref[...] = z2
    g_ref[...] = jnp.mean(z2, axis=0, keepdims=True)


_fin = pl.pallas_call(
    _fin_body,
    out_shape=[
        jax.ShapeDtypeStruct((N, D), jnp.float32),
        jax.ShapeDtypeStruct((1, D), jnp.float32),
    ],
)


def kernel(x, edge_index, W1, b1, W2, b2):
    npad = EP - E
    srcf = jnp.concatenate([edge_index[0], jnp.zeros((npad,), jnp.int32)])
    pad_dst = N + (jnp.arange(npad, dtype=jnp.int32) % (NP - N))
    dstf = jnp.concatenate([edge_index[1], pad_dst])
    src = srcf.reshape(NW, NSB, SB, C)
    dst = dstf.reshape(NW, NSB, SB, C)
    dst2 = dstf.reshape(NW, NCH, C)
    zD = jnp.zeros((RPT, D), jnp.float32)
    onesW = jnp.ones((C, DW), jnp.float32)
    zW = jnp.zeros((RPT, DW), jnp.float32)
    degf = _deg_kernel(dst2, onesW, zW)
    degp = degf[:, :N, 0:1]
    xs = _prescale(degp, x)
    p = _agg_kernel(xs, src, dst, zD)
    zs1 = _mid(degp, p, W1, b1.reshape(1, D))
    q = _agg_kernel(zs1, src, dst, zD)
    z2, g2 = _fin(degp, q, W2, b2.reshape(1, D))
    return (z2, g2)


# speedup vs baseline: 3.0727x; 3.0727x over previous
import functools

import jax
import jax.numpy as jnp
from jax import lax
from jax.experimental import pallas as pl
from jax.experimental.pallas import tpu as pltpu
from jax.experimental.pallas import tpu_sc as plsc

N = 10000
E = 320000
D = 128
NC = 2
NS = 16
NW = NC * NS
NP = 10240
RPT = NP // NS
EPW = E // NW
C = 80
NCH = EPW // C
SB = 25
NSB = NCH // SB

_mesh = plsc.VectorSubcoreMesh(core_axis_name="c", subcore_axis_name="s")


DW = 16


@functools.partial(
    pl.kernel,
    out_type=jax.ShapeDtypeStruct((NC, NP, DW), jnp.float32),
    mesh=_mesh,
    compiler_params=pltpu.CompilerParams(use_tc_tiling_on_sc=False),
    scratch_types=[
        pltpu.VMEM((NCH, C), jnp.int32),
        pltpu.VMEM((C, DW), jnp.float32),
        pltpu.VMEM_SHARED((NP, DW), jnp.float32),
    ],
)
def _deg_kernel(dst_hbm, ones_hbm, zeros_hbm, out_hbm, idx_v, ones_v, deg_sh):
    cid = lax.axis_index("c")
    sid = lax.axis_index("s")
    wid = sid * NC + cid
    pltpu.sync_copy(dst_hbm.at[wid], idx_v)
    pltpu.sync_copy(ones_hbm, ones_v)
    pltpu.sync_copy(zeros_hbm, deg_sh.at[pl.ds(sid * RPT, RPT)])
    plsc.subcore_barrier()

    def body(j, carry):
        pltpu.sync_copy(ones_v, deg_sh.at[idx_v.at[j]], add=True)
        return carry

    lax.fori_loop(0, NCH, body, 0)
    plsc.subcore_barrier()
    pltpu.sync_copy(deg_sh.at[pl.ds(sid * RPT, RPT)],
                    out_hbm.at[cid, pl.ds(sid * RPT, RPT)])


@functools.partial(
    pl.kernel,
    out_type=jax.ShapeDtypeStruct((NC, NP, D), jnp.float32),
    mesh=_mesh,
    scratch_types=[
        pltpu.VMEM((SB, C), jnp.int32),
        pltpu.VMEM((SB, C), jnp.int32),
        pltpu.VMEM((C, D), jnp.float32),
        pltpu.VMEM((C, D), jnp.float32),
        pltpu.VMEM_SHARED((NP, D), jnp.float32),
        pltpu.SemaphoreType.DMA,
        pltpu.SemaphoreType.DMA,
    ],
)
def _agg_kernel(feat_hbm, src_hbm, dst_hbm, zeros_hbm, out_hbm,
                src_v, dst_v, rows0_v, rows1_v, agg_sh, sem0, sem1):
    cid = lax.axis_index("c")
    sid = lax.axis_index("s")
    wid = sid * NC + cid
    pltpu.sync_copy(zeros_hbm, agg_sh.at[pl.ds(sid * RPT, RPT)])
    plsc.subcore_barrier()

    def step(j, rows_v, sem):
        pltpu.make_async_copy(feat_hbm.at[src_v.at[j]], rows_v, sem).wait()
        pltpu.sync_copy(rows_v, agg_sh.at[dst_v.at[j]], add=True)

        @pl.when(j + 2 < SB)
        def _():
            pltpu.async_copy(feat_hbm.at[src_v.at[j + 2]], rows_v, sem)

    def inner(j, carry):
        @pl.when(j % 2 == 0)
        def _():
            step(j, rows0_v, sem0)

        @pl.when(j % 2 == 1)
        def _():
            step(j, rows1_v, sem1)

        return carry

    def outer(s, carry):
        pltpu.sync_copy(src_hbm.at[wid, s], src_v)
        pltpu.sync_copy(dst_hbm.at[wid, s], dst_v)
        pltpu.async_copy(feat_hbm.at[src_v.at[0]], rows0_v, sem0)
        pltpu.async_copy(feat_hbm.at[src_v.at[1]], rows1_v, sem1)
        lax.fori_loop(0, SB, inner, 0)
        return carry

    lax.fori_loop(0, NSB, outer, 0)
    plsc.subcore_barrier()
    pltpu.sync_copy(agg_sh.at[pl.ds(sid * RPT, RPT)],
                    out_hbm.at[cid, pl.ds(sid * RPT, RPT)])


def _dinv_from(degp_ref):
    deg = degp_ref[0] + degp_ref[1]
    dinv = lax.rsqrt(jnp.maximum(deg, 1.0))
    return jnp.where(deg > 0, dinv, 0.0)


def _prescale_body(degp_ref, x_ref, xs_ref):
    xs_ref[...] = x_ref[...] * _dinv_from(degp_ref)


_prescale = pl.pallas_call(
    _prescale_body,
    out_shape=jax.ShapeDtypeStruct((N, D), jnp.float32),
)


def _mid_body(degp_ref, p_ref, w_ref, b_ref, out_ref):
    dinv = _dinv_from(degp_ref)
    agg = (p_ref[0][:N] + p_ref[1][:N]) * dinv
    z1 = jnp.dot(agg, w_ref[...], preferred_element_type=jnp.float32) + b_ref[...]
    out_ref[...] = jnp.maximum(z1, 0.0) * dinv


_mid = pl.pallas_call(
    _mid_body,
    out_shape=jax.ShapeDtypeStruct((N, D), jnp.float32),
)


def _fin_body(degp_ref, q_ref, w_ref, b_ref, z_ref, g_ref):
    dinv = _dinv_from(degp_ref)
    agg = (q_ref[0][:N] + q_ref[1][:N]) * dinv
    z2 = jnp.dot(agg, w_ref[...], preferred_element_type=jnp.float32) + b_ref[...]
    z_ref[...] = z2
    g_ref[...] = jnp.mean(z2, axis=0, keepdims=True)


_fin = pl.pallas_call(
    _fin_body,
    out_shape=[
        jax.ShapeDtypeStruct((N, D), jnp.float32),
        jax.ShapeDtypeStruct((1, D), jnp.float32),
    ],
)


def kernel(x, edge_index, W1, b1, W2, b2):
    src = edge_index[0].reshape(NW, NSB, SB, C)
    dst = edge_index[1].reshape(NW, NSB, SB, C)
    dst2 = edge_index[1].reshape(NW, NCH, C)
    zD = jnp.zeros((RPT, D), jnp.float32)
    onesW = jnp.ones((C, DW), jnp.float32)
    zW = jnp.zeros((RPT, DW), jnp.float32)
    degf = _deg_kernel(dst2, onesW, zW)
    degp = degf[:, :N, 0:1]
    xs = _prescale(degp, x)
    p = _agg_kernel(xs, src, dst, zD)
    zs1 = _mid(degp, p, W1, b1.reshape(1, D))
    q = _agg_kernel(zs1, src, dst, zD)
    z2, g2 = _fin(degp, q, W2, b2.reshape(1, D))
    return (z2, g2)


# speedup vs baseline: 3.0930x; 1.0066x over previous
import functools

import jax
import jax.numpy as jnp
from jax import lax
from jax.experimental import pallas as pl
from jax.experimental.pallas import tpu as pltpu
from jax.experimental.pallas import tpu_sc as plsc

N = 10000
E = 320000
D = 128
NC = 2
NS = 16
NW = NC * NS
NP = 10240
RPT = NP // NS
EPW = E // NW
C = 80
NCH = EPW // C
SB = 25
NSB = NCH // SB

_mesh = plsc.VectorSubcoreMesh(core_axis_name="c", subcore_axis_name="s")


DW = 16


@functools.partial(
    pl.kernel,
    out_type=jax.ShapeDtypeStruct((NC, NP, DW), jnp.float32),
    mesh=_mesh,
    compiler_params=pltpu.CompilerParams(use_tc_tiling_on_sc=False),
    scratch_types=[
        pltpu.VMEM((NCH, C), jnp.int32),
        pltpu.VMEM((C, DW), jnp.float32),
        pltpu.VMEM_SHARED((NP, DW), jnp.float32),
    ],
)
def _deg_kernel(dst_hbm, ones_hbm, zeros_hbm, out_hbm, idx_v, ones_v, deg_sh):
    cid = lax.axis_index("c")
    sid = lax.axis_index("s")
    wid = sid * NC + cid
    pltpu.sync_copy(dst_hbm.at[wid], idx_v)
    pltpu.sync_copy(ones_hbm, ones_v)
    pltpu.sync_copy(zeros_hbm, deg_sh.at[pl.ds(sid * RPT, RPT)])
    plsc.subcore_barrier()

    def body(j, carry):
        pltpu.sync_copy(ones_v, deg_sh.at[idx_v.at[j]], add=True)
        return carry

    lax.fori_loop(0, NCH, body, 0)
    plsc.subcore_barrier()
    pltpu.sync_copy(deg_sh.at[pl.ds(sid * RPT, RPT)],
                    out_hbm.at[cid, pl.ds(sid * RPT, RPT)])


@functools.partial(
    pl.kernel,
    out_type=jax.ShapeDtypeStruct((NC, NP, D), jnp.float32),
    mesh=_mesh,
    compiler_params=pltpu.CompilerParams(use_tc_tiling_on_sc=False),
    scratch_types=[
        pltpu.VMEM((SB, C), jnp.int32),
        pltpu.VMEM((SB, C), jnp.int32),
        pltpu.VMEM((C, D), jnp.float32),
        pltpu.VMEM((C, D), jnp.float32),
        pltpu.VMEM_SHARED((NP, D), jnp.float32),
        pltpu.SemaphoreType.DMA,
        pltpu.SemaphoreType.DMA,
    ],
)
def _agg_kernel(feat_hbm, src_hbm, dst_hbm, zeros_hbm, out_hbm,
                src_v, dst_v, rows0_v, rows1_v, agg_sh, sem0, sem1):
    cid = lax.axis_index("c")
    sid = lax.axis_index("s")
    wid = sid * NC + cid
    pltpu.sync_copy(zeros_hbm, agg_sh.at[pl.ds(sid * RPT, RPT)])
    plsc.subcore_barrier()

    def step(j, rows_v, sem):
        pltpu.make_async_copy(feat_hbm.at[src_v.at[j]], rows_v, sem).wait()
        pltpu.sync_copy(rows_v, agg_sh.at[dst_v.at[j]], add=True)

        @pl.when(j + 2 < SB)
        def _():
            pltpu.async_copy(feat_hbm.at[src_v.at[j + 2]], rows_v, sem)

    def inner(j, carry):
        @pl.when(j % 2 == 0)
        def _():
            step(j, rows0_v, sem0)

        @pl.when(j % 2 == 1)
        def _():
            step(j, rows1_v, sem1)

        return carry

    def outer(s, carry):
        pltpu.sync_copy(src_hbm.at[wid, s], src_v)
        pltpu.sync_copy(dst_hbm.at[wid, s], dst_v)
        pltpu.async_copy(feat_hbm.at[src_v.at[0]], rows0_v, sem0)
        pltpu.async_copy(feat_hbm.at[src_v.at[1]], rows1_v, sem1)
        lax.fori_loop(0, SB, inner, 0)
        return carry

    lax.fori_loop(0, NSB, outer, 0)
    plsc.subcore_barrier()
    pltpu.sync_copy(agg_sh.at[pl.ds(sid * RPT, RPT)],
                    out_hbm.at[cid, pl.ds(sid * RPT, RPT)])


def _dinv_from(degp_ref):
    deg = degp_ref[0] + degp_ref[1]
    dinv = lax.rsqrt(jnp.maximum(deg, 1.0))
    return jnp.where(deg > 0, dinv, 0.0)


def _prescale_body(degp_ref, x_ref, xs_ref):
    xs_ref[...] = x_ref[...] * _dinv_from(degp_ref)


_prescale = pl.pallas_call(
    _prescale_body,
    out_shape=jax.ShapeDtypeStruct((N, D), jnp.float32),
)


def _mid_body(degp_ref, p_ref, w_ref, b_ref, out_ref):
    dinv = _dinv_from(degp_ref)
    agg = (p_ref[0][:N] + p_ref[1][:N]) * dinv
    z1 = jnp.dot(agg, w_ref[...], preferred_element_type=jnp.float32) + b_ref[...]
    out_ref[...] = jnp.maximum(z1, 0.0) * dinv


_mid = pl.pallas_call(
    _mid_body,
    out_shape=jax.ShapeDtypeStruct((N, D), jnp.float32),
)


def _fin_body(degp_ref, q_ref, w_ref, b_ref, z_ref, g_ref):
    dinv = _dinv_from(degp_ref)
    agg = (q_ref[0][:N] + q_ref[1][:N]) * dinv
    z2 = jnp.dot(agg, w_ref[...], preferred_element_type=jnp.float32) + b_ref[...]
    z_ref[...] = z2
    g_ref[...] = jnp.mean(z2, axis=0, keepdims=True)


_fin = pl.pallas_call(
    _fin_body,
    out_shape=[
        jax.ShapeDtypeStruct((N, D), jnp.float32),
        jax.ShapeDtypeStruct((1, D), jnp.float32),
    ],
)


def kernel(x, edge_index, W1, b1, W2, b2):
    src = edge_index[0].reshape(NW, NSB, SB, C)
    dst = edge_index[1].reshape(NW, NSB, SB, C)
    dst2 = edge_index[1].reshape(NW, NCH, C)
    zD = jnp.zeros((RPT, D), jnp.float32)
    onesW = jnp.ones((C, DW), jnp.float32)
    zW = jnp.zeros((RPT, DW), jnp.float32)
    degf = _deg_kernel(dst2, onesW, zW)
    degp = degf[:, :N, 0:1]
    xs = _prescale(degp, x)
    p = _agg_kernel(xs, src, dst, zD)
    zs1 = _mid(degp, p, W1, b1.reshape(1, D))
    q = _agg_kernel(zs1, src, dst, zD)
    z2, g2 = _fin(degp, q, W2, b2.reshape(1, D))
    return (z2, g2)


# speedup vs baseline: 3.4806x; 1.1253x over previous
import functools

import jax
import jax.numpy as jnp
from jax import lax
from jax.experimental import pallas as pl
from jax.experimental.pallas import tpu as pltpu
from jax.experimental.pallas import tpu_sc as plsc

N = 10000
E = 320000
D = 128
NC = 2
NS = 16
NW = NC * NS
NP = 10240
RPT = NP // NS
EPW = E // NW
C = 80
NCH = EPW // C
SB = 25
NSB = NCH // SB

_mesh = plsc.VectorSubcoreMesh(core_axis_name="c", subcore_axis_name="s")


DW = 16


@functools.partial(
    pl.kernel,
    out_type=jax.ShapeDtypeStruct((NC, NP, DW), jnp.float32),
    mesh=_mesh,
    compiler_params=pltpu.CompilerParams(use_tc_tiling_on_sc=False),
    scratch_types=[
        pltpu.VMEM((NCH, C), jnp.int32),
        pltpu.VMEM((C, DW), jnp.float32),
        pltpu.VMEM_SHARED((NP, DW), jnp.float32),
    ],
)
def _deg_kernel(dst_hbm, ones_hbm, zeros_hbm, out_hbm, idx_v, ones_v, deg_sh):
    cid = lax.axis_index("c")
    sid = lax.axis_index("s")
    wid = sid * NC + cid
    pltpu.sync_copy(dst_hbm.at[wid], idx_v)
    pltpu.sync_copy(ones_hbm, ones_v)
    pltpu.sync_copy(zeros_hbm, deg_sh.at[pl.ds(sid * RPT, RPT)])
    plsc.subcore_barrier()

    def body(j, carry):
        pltpu.sync_copy(ones_v, deg_sh.at[idx_v.at[j]], add=True)
        return carry

    lax.fori_loop(0, NCH, body, 0)
    plsc.subcore_barrier()
    pltpu.sync_copy(deg_sh.at[pl.ds(sid * RPT, RPT)],
                    out_hbm.at[cid, pl.ds(sid * RPT, RPT)])


@functools.partial(
    pl.kernel,
    out_type=jax.ShapeDtypeStruct((NC, NP, D), jnp.float32),
    mesh=_mesh,
    compiler_params=pltpu.CompilerParams(use_tc_tiling_on_sc=False),
    scratch_types=[
        pltpu.VMEM((SB, C), jnp.int32),
        pltpu.VMEM((SB, C), jnp.int32),
        pltpu.VMEM((C, D), jnp.float32),
        pltpu.VMEM((C, D), jnp.float32),
        pltpu.VMEM((C, D), jnp.float32),
        pltpu.VMEM_SHARED((NP, D), jnp.float32),
        pltpu.SemaphoreType.DMA,
        pltpu.SemaphoreType.DMA,
        pltpu.SemaphoreType.DMA,
    ],
)
def _agg_kernel(feat_hbm, src_hbm, dst_hbm, zeros_hbm, out_hbm,
                src_v, dst_v, rows0_v, rows1_v, rows2_v, agg_sh,
                sem0, sem1, sem2):
    cid = lax.axis_index("c")
    sid = lax.axis_index("s")
    wid = sid * NC + cid
    pltpu.sync_copy(zeros_hbm, agg_sh.at[pl.ds(sid * RPT, RPT)])
    plsc.subcore_barrier()

    def step(j, rows_v, sem):
        pltpu.make_async_copy(feat_hbm.at[src_v.at[j]], rows_v, sem).wait()
        pltpu.sync_copy(rows_v, agg_sh.at[dst_v.at[j]], add=True)

        @pl.when(j + 3 < SB)
        def _():
            pltpu.async_copy(feat_hbm.at[src_v.at[j + 3]], rows_v, sem)

    def inner(j, carry):
        @pl.when(j % 3 == 0)
        def _():
            step(j, rows0_v, sem0)

        @pl.when(j % 3 == 1)
        def _():
            step(j, rows1_v, sem1)

        @pl.when(j % 3 == 2)
        def _():
            step(j, rows2_v, sem2)

        return carry

    def outer(s, carry):
        pltpu.sync_copy(src_hbm.at[wid, s], src_v)
        pltpu.sync_copy(dst_hbm.at[wid, s], dst_v)
        pltpu.async_copy(feat_hbm.at[src_v.at[0]], rows0_v, sem0)
        pltpu.async_copy(feat_hbm.at[src_v.at[1]], rows1_v, sem1)
        pltpu.async_copy(feat_hbm.at[src_v.at[2]], rows2_v, sem2)
        lax.fori_loop(0, SB, inner, 0)
        return carry

    lax.fori_loop(0, NSB, outer, 0)
    plsc.subcore_barrier()
    pltpu.sync_copy(agg_sh.at[pl.ds(sid * RPT, RPT)],
                    out_hbm.at[cid, pl.ds(sid * RPT, RPT)])


def _dinv_from(degp_ref):
    deg = degp_ref[0] + degp_ref[1]
    dinv = lax.rsqrt(jnp.maximum(deg, 1.0))
    return jnp.where(deg > 0, dinv, 0.0)


def _prescale_body(degp_ref, x_ref, xs_ref):
    xs_ref[...] = x_ref[...] * _dinv_from(degp_ref)


_prescale = pl.pallas_call(
    _prescale_body,
    out_shape=jax.ShapeDtypeStruct((N, D), jnp.float32),
)


def _mid_body(degp_ref, p_ref, w_ref, b_ref, out_ref):
    dinv = _dinv_from(degp_ref)
    agg = (p_ref[0][:N] + p_ref[1][:N]) * dinv
    z1 = jnp.dot(agg, w_ref[...], preferred_element_type=jnp.float32) + b_ref[...]
    out_ref[...] = jnp.maximum(z1, 0.0) * dinv


_mid = pl.pallas_call(
    _mid_body,
    out_shape=jax.ShapeDtypeStruct((N, D), jnp.float32),
)


def _fin_body(degp_ref, q_ref, w_ref, b_ref, z_ref, g_ref):
    dinv = _dinv_from(degp_ref)
    agg = (q_ref[0][:N] + q_ref[1][:N]) * dinv
    z2 = jnp.dot(agg, w_ref[...], preferred_element_type=jnp.float32) + b_ref[...]
    z_ref[...] = z2
    g_ref[...] = jnp.mean(z2, axis=0, keepdims=True)


_fin = pl.pallas_call(
    _fin_body,
    out_shape=[
        jax.ShapeDtypeStruct((N, D), jnp.float32),
        jax.ShapeDtypeStruct((1, D), jnp.float32),
    ],
)


def kernel(x, edge_index, W1, b1, W2, b2):
    src = edge_index[0].reshape(NW, NSB, SB, C)
    dst = edge_index[1].reshape(NW, NSB, SB, C)
    dst2 = edge_index[1].reshape(NW, NCH, C)
    zD = jnp.zeros((RPT, D), jnp.float32)
    onesW = jnp.ones((C, DW), jnp.float32)
    zW = jnp.zeros((RPT, DW), jnp.float32)
    degf = _deg_kernel(dst2, onesW, zW)
    degp = degf[:, :N, 0:1]
    xs = _prescale(degp, x)
    p = _agg_kernel(xs, src, dst, zD)
    zs1 = _mid(degp, p, W1, b1.reshape(1, D))
    q = _agg_kernel(zs1, src, dst, zD)
    z2, g2 = _fin(degp, q, W2, b2.reshape(1, D))
    return (z2, g2)


# speedup vs baseline: 3.4809x; 1.0001x over previous
import functools

import jax
import jax.numpy as jnp
from jax import lax
from jax.experimental import pallas as pl
from jax.experimental.pallas import tpu as pltpu
from jax.experimental.pallas import tpu_sc as plsc

N = 10000
E = 320000
D = 128
NC = 2
NS = 16
NW = NC * NS
NP = 10240
RPT = NP // NS
EPW = E // NW
C = 80
NCH = EPW // C
SB = 25
NSB = NCH // SB

_mesh = plsc.VectorSubcoreMesh(core_axis_name="c", subcore_axis_name="s")


DW = 16


@functools.partial(
    pl.kernel,
    out_type=jax.ShapeDtypeStruct((NC, NP, DW), jnp.float32),
    mesh=_mesh,
    compiler_params=pltpu.CompilerParams(use_tc_tiling_on_sc=False),
    scratch_types=[
        pltpu.VMEM((NCH, C), jnp.int32),
        pltpu.VMEM((C, DW), jnp.float32),
        pltpu.VMEM_SHARED((NP, DW), jnp.float32),
    ],
)
def _deg_kernel(dst_hbm, ones_hbm, zeros_hbm, out_hbm, idx_v, ones_v, deg_sh):
    cid = lax.axis_index("c")
    sid = lax.axis_index("s")
    wid = sid * NC + cid
    pltpu.sync_copy(dst_hbm.at[wid], idx_v)
    pltpu.sync_copy(ones_hbm, ones_v)
    pltpu.sync_copy(zeros_hbm, deg_sh.at[pl.ds(sid * RPT, RPT)])
    plsc.subcore_barrier()

    def body(j, carry):
        pltpu.sync_copy(ones_v, deg_sh.at[idx_v.at[j]], add=True)
        return carry

    lax.fori_loop(0, NCH, body, 0)
    plsc.subcore_barrier()
    pltpu.sync_copy(deg_sh.at[pl.ds(sid * RPT, RPT)],
                    out_hbm.at[cid, pl.ds(sid * RPT, RPT)])


@functools.partial(
    pl.kernel,
    out_type=jax.ShapeDtypeStruct((NC, NP, D), jnp.float32),
    mesh=_mesh,
    compiler_params=pltpu.CompilerParams(use_tc_tiling_on_sc=False),
    scratch_types=[
        pltpu.VMEM((SB, C), jnp.int32),
        pltpu.VMEM((SB, C), jnp.int32),
        pltpu.VMEM((C, D), jnp.float32),
        pltpu.VMEM((C, D), jnp.float32),
        pltpu.VMEM((C, D), jnp.float32),
        pltpu.VMEM_SHARED((NP, D), jnp.float32),
        pltpu.SemaphoreType.DMA,
        pltpu.SemaphoreType.DMA,
        pltpu.SemaphoreType.DMA,
        pltpu.SemaphoreType.DMA,
        pltpu.SemaphoreType.DMA,
        pltpu.SemaphoreType.DMA,
    ],
)
def _agg_kernel(feat_hbm, src_hbm, dst_hbm, zeros_hbm, out_hbm,
                src_v, dst_v, rows0_v, rows1_v, rows2_v, agg_sh,
                sem0, sem1, sem2, ssem0, ssem1, ssem2):
    cid = lax.axis_index("c")
    sid = lax.axis_index("s")
    wid = sid * NC + cid
    pltpu.sync_copy(zeros_hbm, agg_sh.at[pl.ds(sid * RPT, RPT)])
    plsc.subcore_barrier()

    def _drain(rows_v, ssem):
        pltpu.make_async_copy(rows_v, agg_sh.at[dst_v.at[0]], ssem).wait()

    def step(j, rows_v, sem, ssem):
        pltpu.make_async_copy(feat_hbm.at[src_v.at[j]], rows_v, sem).wait()
        pltpu.async_copy(rows_v, agg_sh.at[dst_v.at[j]], ssem, add=True)

        @pl.when(j + 3 < SB)
        def _():
            _drain(rows_v, ssem)
            pltpu.async_copy(feat_hbm.at[src_v.at[j + 3]], rows_v, sem)

    def inner(j, carry):
        @pl.when(j % 3 == 0)
        def _():
            step(j, rows0_v, sem0, ssem0)

        @pl.when(j % 3 == 1)
        def _():
            step(j, rows1_v, sem1, ssem1)

        @pl.when(j % 3 == 2)
        def _():
            step(j, rows2_v, sem2, ssem2)

        return carry

    def outer(s, carry):
        @pl.when(s > 0)
        def _():
            _drain(rows0_v, ssem0)
            _drain(rows1_v, ssem1)
            _drain(rows2_v, ssem2)

        pltpu.sync_copy(src_hbm.at[wid, s], src_v)
        pltpu.sync_copy(dst_hbm.at[wid, s], dst_v)
        pltpu.async_copy(feat_hbm.at[src_v.at[0]], rows0_v, sem0)
        pltpu.async_copy(feat_hbm.at[src_v.at[1]], rows1_v, sem1)
        pltpu.async_copy(feat_hbm.at[src_v.at[2]], rows2_v, sem2)
        lax.fori_loop(0, SB, inner, 0)
        return carry

    lax.fori_loop(0, NSB, outer, 0)
    _drain(rows0_v, ssem0)
    _drain(rows1_v, ssem1)
    _drain(rows2_v, ssem2)
    plsc.subcore_barrier()
    pltpu.sync_copy(agg_sh.at[pl.ds(sid * RPT, RPT)],
                    out_hbm.at[cid, pl.ds(sid * RPT, RPT)])


def _dinv_from(degp_ref):
    deg = degp_ref[0] + degp_ref[1]
    dinv = lax.rsqrt(jnp.maximum(deg, 1.0))
    return jnp.where(deg > 0, dinv, 0.0)


def _prescale_body(degp_ref, x_ref, xs_ref):
    xs_ref[...] = x_ref[...] * _dinv_from(degp_ref)


_prescale = pl.pallas_call(
    _prescale_body,
    out_shape=jax.ShapeDtypeStruct((N, D), jnp.float32),
)


def _mid_body(degp_ref, p_ref, w_ref, b_ref, out_ref):
    dinv = _dinv_from(degp_ref)
    agg = (p_ref[0][:N] + p_ref[1][:N]) * dinv
    z1 = jnp.dot(agg, w_ref[...], preferred_element_type=jnp.float32) + b_ref[...]
    out_ref[...] = jnp.maximum(z1, 0.0) * dinv


_mid = pl.pallas_call(
    _mid_body,
    out_shape=jax.ShapeDtypeStruct((N, D), jnp.float32),
)


def _fin_body(degp_ref, q_ref, w_ref, b_ref, z_ref, g_ref):
    dinv = _dinv_from(degp_ref)
    agg = (q_ref[0][:N] + q_ref[1][:N]) * dinv
    z2 = jnp.dot(agg, w_ref[...], preferred_element_type=jnp.float32) + b_ref[...]
    z_ref[...] = z2
    g_ref[...] = jnp.mean(z2, axis=0, keepdims=True)


_fin = pl.pallas_call(
    _fin_body,
    out_shape=[
        jax.ShapeDtypeStruct((N, D), jnp.float32),
        jax.ShapeDtypeStruct((1, D), jnp.float32),
    ],
)


def kernel(x, edge_index, W1, b1, W2, b2):
    src = edge_index[0].reshape(NW, NSB, SB, C)
    dst = edge_index[1].reshape(NW, NSB, SB, C)
    dst2 = edge_index[1].reshape(NW, NCH, C)
    zD = jnp.zeros((RPT, D), jnp.float32)
    onesW = jnp.ones((C, DW), jnp.float32)
    zW = jnp.zeros((RPT, DW), jnp.float32)
    degf = _deg_kernel(dst2, onesW, zW)
    degp = degf[:, :N, 0:1]
    xs = _prescale(degp, x)
    p = _agg_kernel(xs, src, dst, zD)
    zs1 = _mid(degp, p, W1, b1.reshape(1, D))
    q = _agg_kernel(zs1, src, dst, zD)
    z2, g2 = _fin(degp, q, W2, b2.reshape(1, D))
    return (z2, g2)


# speedup vs baseline: 3.4826x; 1.0005x over previous
import functools

import jax
import jax.numpy as jnp
from jax import lax
from jax.experimental import pallas as pl
from jax.experimental.pallas import tpu as pltpu
from jax.experimental.pallas import tpu_sc as plsc

N = 10000
E = 320000
D = 128
NC = 2
NS = 16
NW = NC * NS
NP = 10240
RPT = NP // NS
EPW = E // NW
C = 80
NCH = EPW // C
SB = 25
NSB = NCH // SB

_mesh = plsc.VectorSubcoreMesh(core_axis_name="c", subcore_axis_name="s")


DW = 16


@functools.partial(
    pl.kernel,
    out_type=jax.ShapeDtypeStruct((NC, NP, DW), jnp.float32),
    mesh=_mesh,
    compiler_params=pltpu.CompilerParams(use_tc_tiling_on_sc=False),
    scratch_types=[
        pltpu.VMEM((NCH, C), jnp.int32),
        pltpu.VMEM((C, DW), jnp.float32),
        pltpu.VMEM_SHARED((NP, DW), jnp.float32),
    ],
)
def _deg_kernel(dst_hbm, ones_hbm, zeros_hbm, out_hbm, idx_v, ones_v, deg_sh):
    cid = lax.axis_index("c")
    sid = lax.axis_index("s")
    wid = sid * NC + cid
    pltpu.sync_copy(dst_hbm.at[wid], idx_v)
    pltpu.sync_copy(ones_hbm, ones_v)
    pltpu.sync_copy(zeros_hbm, deg_sh.at[pl.ds(sid * RPT, RPT)])
    plsc.subcore_barrier()

    def body(j, carry):
        pltpu.sync_copy(ones_v, deg_sh.at[idx_v.at[j]], add=True)
        return carry

    lax.fori_loop(0, NCH, body, 0)
    plsc.subcore_barrier()
    pltpu.sync_copy(deg_sh.at[pl.ds(sid * RPT, RPT)],
                    out_hbm.at[cid, pl.ds(sid * RPT, RPT)])


@functools.partial(
    pl.kernel,
    out_type=jax.ShapeDtypeStruct((NC, NP, D), jnp.float32),
    mesh=_mesh,
    compiler_params=pltpu.CompilerParams(use_tc_tiling_on_sc=False),
    scratch_types=[
        pltpu.VMEM((SB, C), jnp.int32),
        pltpu.VMEM((SB, C), jnp.int32),
        pltpu.VMEM((C, D), jnp.float32),
        pltpu.VMEM((C, D), jnp.float32),
        pltpu.VMEM((C, D), jnp.float32),
        pltpu.VMEM_SHARED((NP, D), jnp.float32),
        pltpu.SemaphoreType.DMA,
        pltpu.SemaphoreType.DMA,
        pltpu.SemaphoreType.DMA,
    ],
)
def _agg_kernel(feat_hbm, src_hbm, dst_hbm, zeros_hbm, out_hbm,
                src_v, dst_v, rows0_v, rows1_v, rows2_v, agg_sh,
                sem0, sem1, sem2):
    cid = lax.axis_index("c")
    sid = lax.axis_index("s")
    wid = sid * NC + cid
    pltpu.sync_copy(zeros_hbm, agg_sh.at[pl.ds(sid * RPT, RPT)])
    plsc.subcore_barrier()

    def step(j, rows_v, sem):
        pltpu.make_async_copy(feat_hbm.at[src_v.at[j]], rows_v, sem).wait()
        pltpu.sync_copy(rows_v, agg_sh.at[dst_v.at[j]], add=True)

        @pl.when(j + 3 < SB)
        def _():
            pltpu.async_copy(feat_hbm.at[src_v.at[j + 3]], rows_v, sem)

    def inner(j, carry):
        @pl.when(j % 3 == 0)
        def _():
            step(j, rows0_v, sem0)

        @pl.when(j % 3 == 1)
        def _():
            step(j, rows1_v, sem1)

        @pl.when(j % 3 == 2)
        def _():
            step(j, rows2_v, sem2)

        return carry

    def outer(s, carry):
        pltpu.sync_copy(src_hbm.at[wid, s], src_v)
        pltpu.sync_copy(dst_hbm.at[wid, s], dst_v)
        pltpu.async_copy(feat_hbm.at[src_v.at[0]], rows0_v, sem0)
        pltpu.async_copy(feat_hbm.at[src_v.at[1]], rows1_v, sem1)
        pltpu.async_copy(feat_hbm.at[src_v.at[2]], rows2_v, sem2)
        lax.fori_loop(0, SB, inner, 0)
        return carry

    lax.fori_loop(0, NSB, outer, 0)
    plsc.subcore_barrier()
    pltpu.sync_copy(agg_sh.at[pl.ds(sid * RPT, RPT)],
                    out_hbm.at[cid, pl.ds(sid * RPT, RPT)])


def _dinv_from(degp_ref):
    deg = degp_ref[0] + degp_ref[1]
    dinv = lax.rsqrt(jnp.maximum(deg, 1.0))
    return jnp.where(deg > 0, dinv, 0.0)


def _prescale_body(degp_ref, x_ref, xs_ref):
    xs_ref[...] = x_ref[...] * _dinv_from(degp_ref)


_prescale = pl.pallas_call(
    _prescale_body,
    out_shape=jax.ShapeDtypeStruct((N, D), jnp.float32),
)


def _mid_body(degp_ref, p_ref, w_ref, b_ref, out_ref):
    dinv = _dinv_from(degp_ref)
    agg = (p_ref[0][:N] + p_ref[1][:N]) * dinv
    z1 = jnp.dot(agg, w_ref[...], preferred_element_type=jnp.float32) + b_ref[...]
    out_ref[...] = jnp.maximum(z1, 0.0) * dinv


_mid = pl.pallas_call(
    _mid_body,
    out_shape=jax.ShapeDtypeStruct((N, D), jnp.float32),
)


def _fin_body(degp_ref, q_ref, w_ref, b_ref, z_ref, g_ref):
    dinv = _dinv_from(degp_ref)
    agg = (q_ref[0][:N] + q_ref[1][:N]) * dinv
    z2 = jnp.dot(agg, w_ref[...], preferred_element_type=jnp.float32) + b_ref[...]
    z_ref[...] = z2
    g_ref[...] = jnp.mean(z2, axis=0, keepdims=True)


_fin = pl.pallas_call(
    _fin_body,
    out_shape=[
        jax.ShapeDtypeStruct((N, D), jnp.float32),
        jax.ShapeDtypeStruct((1, D), jnp.float32),
    ],
)


def kernel(x, edge_index, W1, b1, W2, b2):
    src = edge_index[0].reshape(NW, NSB, SB, C)
    dst = edge_index[1].reshape(NW, NSB, SB, C)
    dst2 = edge_index[1].reshape(NW, NCH, C)
    zD = jnp.zeros((RPT, D), jnp.float32)
    onesW = jnp.ones((C, DW), jnp.float32)
    zW = jnp.zeros((RPT, DW), jnp.float32)
    degf = _deg_kernel(dst2, onesW, zW)
    degp = degf[:, :N, 0:1]
    xs = _prescale(degp, x)
    p = _agg_kernel(xs, src, dst, zD)
    zs1 = _mid(degp, p, W1, b1.reshape(1, D))
    q = _agg_kernel(zs1, src, dst, zD)
    z2, g2 = _fin(degp, q, W2, b2.reshape(1, D))
    return (z2, g2)
